# baseline (speedup 1.0000x reference)
import jax
import jax.numpy as jnp
from jax.experimental import pallas as pl
from jax.experimental.pallas import tpu as pltpu


def _rowdot_kernel(gu_ref, gi_ref, out_ref, acc_ref):
    i = pl.program_id(0)
    p = gu_ref[0] * gi_ref[0]
    p4 = p[0:8] + p[8:16]

    @pl.when(i == 0)
    def _init():
        acc_ref[...] = p4

    @pl.when(i != 0)
    def _accum():
        acc_ref[...] += p4

    @pl.when(i == pl.num_programs(0) - 1)
    def _finish():
        out_ref[...] = jnp.sum(acc_ref[...], axis=0)


def kernel(inputs):
    n = inputs.shape[1]
    d = inputs.shape[2]
    t = jnp.swapaxes(inputs, 1, 2)
    kblock = 16
    return pl.pallas_call(
        _rowdot_kernel,
        grid=(d // kblock,),
        in_specs=[
            pl.BlockSpec((1, kblock, n), lambda i: (0, i, 0)),
            pl.BlockSpec((1, kblock, n), lambda i: (1, i, 0)),
        ],
        out_specs=pl.BlockSpec((n,), lambda i: (0,)),
        out_shape=jax.ShapeDtypeStruct((n,), inputs.dtype),
        scratch_shapes=[pltpu.VMEM((8, n), jnp.float32)],
        compiler_params=pltpu.CompilerParams(
            dimension_semantics=("arbitrary",),
        ),
    )(t, t)


# speedup vs baseline: 1.1520x; 1.1520x over previous
import jax
import jax.numpy as jnp
from jax.experimental import pallas as pl
from jax.experimental.pallas import tpu as pltpu


def _rowdot_kernel(gu_ref, gi_ref, out_ref):
    out_ref[...] = jnp.sum(gu_ref[0] * gi_ref[0], axis=0)


def kernel(inputs):
    n = inputs.shape[1]
    d = inputs.shape[2]
    t = jnp.swapaxes(inputs, 1, 2)
    block = 8192
    return pl.pallas_call(
        _rowdot_kernel,
        grid=(n // block,),
        in_specs=[
            pl.BlockSpec((1, d, block), lambda i: (0, 0, i)),
            pl.BlockSpec((1, d, block), lambda i: (1, 0, i)),
        ],
        out_specs=pl.BlockSpec((block,), lambda i: (i,)),
        out_shape=jax.ShapeDtypeStruct((n,), inputs.dtype),
        compiler_params=pltpu.CompilerParams(
            dimension_semantics=("arbitrary",),
        ),
    )(t, t)


# speedup vs baseline: 1.1574x; 1.0047x over previous
import jax
import jax.numpy as jnp
from jax.experimental import pallas as pl
from jax.experimental.pallas import tpu as pltpu


def _rowdot_kernel(x_ref, out_ref):
    out_ref[...] = jnp.sum(x_ref[0] * x_ref[1], axis=0)


def kernel(inputs):
    n = inputs.shape[1]
    d = inputs.shape[2]
    t = jnp.swapaxes(inputs, 1, 2)
    block = 8192
    return pl.pallas_call(
        _rowdot_kernel,
        grid=(n // block,),
        in_specs=[
            pl.BlockSpec((2, d, block), lambda i: (0, 0, i)),
        ],
        out_specs=pl.BlockSpec((block,), lambda i: (i,)),
        out_shape=jax.ShapeDtypeStruct((n,), inputs.dtype),
        compiler_params=pltpu.CompilerParams(
            dimension_semantics=("arbitrary",),
        ),
    )(t)


# speedup vs baseline: 1.1678x; 1.0090x over previous
import jax
import jax.numpy as jnp
from jax.experimental import pallas as pl
from jax.experimental.pallas import tpu as pltpu


def _rowdot_kernel(x_ref, out_ref):
    out_ref[...] = jnp.sum(x_ref[0] * x_ref[1], axis=0)


def kernel(inputs):
    n = inputs.shape[1]
    d = inputs.shape[2]
    t = jnp.swapaxes(inputs, 1, 2)
    block = 8192
    return pl.pallas_call(
        _rowdot_kernel,
        grid=(n // block,),
        in_specs=[
            pl.BlockSpec((2, d, block), lambda i: (0, 0, i)),
        ],
        out_specs=pl.BlockSpec((block,), lambda i: (i,)),
        out_shape=jax.ShapeDtypeStruct((n,), inputs.dtype),
        compiler_params=pltpu.CompilerParams(
            dimension_semantics=("parallel",),
        ),
    )(t)
